# baseline (speedup 1.0000x reference)
import functools

import jax
import jax.numpy as jnp
from jax import lax
from jax.experimental import pallas as pl
from jax.experimental.pallas import tpu as pltpu
from jax.experimental.pallas import tpu_sc as plsc

_NC = 2
_NS = 16
_NW = _NC * _NS


def _sc_gather(x_flat, word_table):
    B = x_flat.shape[0]
    V, D = word_table.shape
    b_per_w = B // _NW
    C = 64
    n_chunks = b_per_w // C

    xr = x_flat.reshape(_NW, n_chunks, C)
    mesh = plsc.VectorSubcoreMesh(core_axis_name="c", subcore_axis_name="s")

    @functools.partial(
        pl.kernel,
        mesh=mesh,
        out_type=jax.ShapeDtypeStruct((B, D), jnp.float32),
        scratch_types=[
            pltpu.VMEM((n_chunks, C), jnp.int32),
            pltpu.VMEM((C, D), jnp.float32),
            pltpu.VMEM((C, D), jnp.float32),
            pltpu.SemaphoreType.DMA,
            pltpu.SemaphoreType.DMA,
        ],
    )
    def k(x_hbm, wt_hbm, out_hbm, idx_v, buf0, buf1, sem0, sem1):
        wid = lax.axis_index("s") * _NC + lax.axis_index("c")
        base = wid * b_per_w
        pltpu.sync_copy(x_hbm.at[wid], idx_v)
        bufs = (buf0, buf1)
        sems = (sem0, sem1)
        copies = [None] * n_chunks
        copies[0] = pltpu.async_copy(wt_hbm.at[idx_v.at[0]], bufs[0], sems[0])
        for c in range(n_chunks):
            copies[c].wait()
            if c + 1 < n_chunks:
                copies[c + 1] = pltpu.async_copy(
                    wt_hbm.at[idx_v.at[c + 1]], bufs[(c + 1) % 2], sems[(c + 1) % 2]
                )
            pltpu.sync_copy(bufs[c % 2], out_hbm.at[pl.ds(base + c * C, C)])

    return k(xr, word_table)


def _tc_epilogue(g, pos_table, x_flat):
    B, D = g.shape
    S = pos_table.shape[0]
    R = 256
    n_blocks = B // R
    pos_blocks = S // R
    factor = float(D) ** 0.5
    x3 = x_flat.reshape(n_blocks, 1, R)

    def body(g_ref, pos_ref, x_ref, out_ref, mask_ref):
        out_ref[...] = g_ref[...] * factor + pos_ref[...]
        mask_ref[...] = x_ref[...] == 0

    out, mask = pl.pallas_call(
        body,
        grid=(n_blocks,),
        in_specs=[
            pl.BlockSpec((R, D), lambda i: (i, 0)),
            pl.BlockSpec((R, D), lambda i: (i % pos_blocks, 0)),
            pl.BlockSpec((1, 1, R), lambda i: (i, 0, 0)),
        ],
        out_specs=[
            pl.BlockSpec((R, D), lambda i: (i, 0)),
            pl.BlockSpec((1, 1, R), lambda i: (i, 0, 0)),
        ],
        out_shape=[
            jax.ShapeDtypeStruct((B, D), jnp.float32),
            jax.ShapeDtypeStruct((n_blocks, 1, R), jnp.bool_),
        ],
    )(g, pos_table, x3)
    return out, mask


def kernel(x, word_table, pos_table):
    batch, seq = x.shape
    D = word_table.shape[1]
    x_flat = x.reshape(-1)
    g = _sc_gather(x_flat, word_table)
    out, mask = _tc_epilogue(g, pos_table, x_flat)
    return out.reshape(batch, seq, D), mask.reshape(batch, seq)
